# baseline (speedup 1.0000x reference)
import functools

import jax
import jax.numpy as jnp
import numpy as np
from jax.experimental import pallas as pl
from jax.experimental.pallas import tpu as pltpu

_ROT0 = (13, 15, 26, 6)
_ROT1 = (17, 29, 16, 24)
_KS0 = np.uint32(0)
_KS1 = np.uint32(42)
_KS2 = np.uint32(0 ^ 42 ^ 0x1BD11BDA)


def _rotl(x, d):
    return (x << np.uint32(d)) | (x >> np.uint32(32 - d))


def _threefry_bits(lin_u32):
    x1 = lin_u32 + _KS1
    x0 = x1

    def rounds(x0, x1, rots, skip_first_add=False):
        for n, r in enumerate(rots):
            if not (skip_first_add and n == 0):
                x0 = x0 + x1
            x1 = _rotl(x1, r) ^ x0
        return x0, x1

    x0, x1 = rounds(x0, x1, _ROT0, skip_first_add=True)
    x0 = x0 + _KS1
    x1 = x1 + np.uint32(_KS2 + np.uint32(1))
    x0, x1 = rounds(x0, x1, _ROT1)
    x0 = x0 + _KS2
    x1 = x1 + np.uint32(_KS0 + np.uint32(2))
    x0, x1 = rounds(x0, x1, _ROT0)
    x1 = x1 + np.uint32(_KS1 + np.uint32(3))
    x0, x1 = rounds(x0, x1, _ROT1)
    x0 = x0 + _KS1
    x1 = x1 + np.uint32(_KS2 + np.uint32(4))
    x0, x1 = rounds(x0, x1, _ROT0)
    x0 = x0 + _KS2
    x1 = x1 + np.uint32(_KS0 + np.uint32(5))
    return x0 ^ x1


_TINY = np.float32(np.finfo(np.float32).tiny)
assert np.float32(1.0) - _TINY == np.float32(1.0)


def _gumbel_from_bits(bits):
    float_bits = (bits >> np.uint32(9)) | np.uint32(0x3F800000)
    f = jax.lax.bitcast_convert_type(float_bits, jnp.float32) - np.float32(1.0)
    u = jnp.maximum(_TINY, f + _TINY)
    return -jnp.log(-jnp.log(u))


def _sample_kernel(logits_ref, out_ref, best_val, best_idx, *, rb, cb, tw,
                   ncols):
    i = pl.program_id(0)
    j = pl.program_id(1)
    ncb = pl.num_programs(1)

    r_iota = jax.lax.broadcasted_iota(jnp.int32, (rb, tw), 0)
    c_iota = jax.lax.broadcasted_iota(jnp.int32, (rb, tw), 1)
    lin_tile = r_iota * ncols + c_iota
    base = i * (rb * ncols) + j * cb
    col0 = j * cb

    rm = jnp.where(j == 0, jnp.full((rb, tw), -jnp.inf, jnp.float32),
                   best_val[...])
    ri = best_idx[...]

    for t in range(cb // tw):
        lin = (lin_tile + (base + t * tw)).astype(jnp.uint32)
        g = _gumbel_from_bits(_threefry_bits(lin))
        vals = g + logits_ref[:, t * tw:(t + 1) * tw]
        c_vec = c_iota + (col0 + t * tw)
        vals = jnp.where(c_vec < ncols, vals, -jnp.inf)
        upd = vals > rm
        rm = jnp.maximum(rm, vals)
        ri = jnp.where(upd, c_vec, ri)

    best_val[...] = rm
    best_idx[...] = ri

    @pl.when(j == ncb - 1)
    def _():
        m = jnp.max(rm, axis=1, keepdims=True)
        cand = jnp.where(rm == m, ri, jnp.int32(np.iinfo(np.int32).max))
        out_ref[...] = jnp.min(cand, axis=1, keepdims=True)


def kernel(logits):
    nrows, ncols = logits.shape
    rb = 64
    cb = 4608
    tw = 256
    grid = (nrows // rb, pl.cdiv(ncols, cb))

    out = pl.pallas_call(
        functools.partial(_sample_kernel, rb=rb, cb=cb, tw=tw, ncols=ncols),
        grid=grid,
        in_specs=[pl.BlockSpec((rb, cb), lambda i, j: (i, j))],
        out_specs=pl.BlockSpec((rb, 1), lambda i, j: (i, 0)),
        out_shape=jax.ShapeDtypeStruct((nrows, 1), jnp.int32),
        scratch_shapes=[
            pltpu.VMEM((rb, tw), jnp.float32),
            pltpu.VMEM((rb, tw), jnp.int32),
        ],
    )(logits)
    return out.reshape(nrows).astype(jnp.int64)


# speedup vs baseline: 1.2673x; 1.2673x over previous
import functools

import jax
import jax.numpy as jnp
import numpy as np
from jax.experimental import pallas as pl
from jax.experimental.pallas import tpu as pltpu

_ROT0 = (13, 15, 26, 6)
_ROT1 = (17, 29, 16, 24)
_KS0 = np.uint32(0)
_KS1 = np.uint32(42)
_KS2 = np.uint32(0 ^ 42 ^ 0x1BD11BDA)


def _rotl(x, d):
    return (x << np.uint32(d)) | (x >> np.uint32(32 - d))


def _threefry_bits(x1_init):
    x1 = x1_init
    x0 = x1

    def rounds(x0, x1, rots, skip_first_add=False):
        for n, r in enumerate(rots):
            if not (skip_first_add and n == 0):
                x0 = x0 + x1
            x1 = _rotl(x1, r) ^ x0
        return x0, x1

    x0, x1 = rounds(x0, x1, _ROT0, skip_first_add=True)
    x0 = x0 + _KS1
    x1 = x1 + np.uint32(_KS2 + np.uint32(1))
    x0, x1 = rounds(x0, x1, _ROT1)
    x0 = x0 + _KS2
    x1 = x1 + np.uint32(_KS0 + np.uint32(2))
    x0, x1 = rounds(x0, x1, _ROT0)
    x1 = x1 + np.uint32(_KS1 + np.uint32(3))
    x0, x1 = rounds(x0, x1, _ROT1)
    x0 = x0 + _KS1
    x1 = x1 + np.uint32(_KS2 + np.uint32(4))
    x0, x1 = rounds(x0, x1, _ROT0)
    x0 = x0 + _KS2
    x1 = x1 + np.uint32(_KS0 + np.uint32(5))
    return x0 ^ x1


_TINY = np.float32(np.finfo(np.float32).tiny)
assert np.float32(1.0) - _TINY == np.float32(1.0)


def _neg_gumbel_from_bits(bits):
    float_bits = (bits >> np.uint32(9)) | np.uint32(0x3F800000)
    f = jax.lax.bitcast_convert_type(float_bits, jnp.float32) - np.float32(1.0)
    u = f + _TINY
    return jnp.log(-jnp.log(u))


def _sample_kernel(logits_ref, out_ref, best_val, best_idx, *, cbv, tws,
                   nbatch, ncols):
    s = pl.program_id(0)
    ns = pl.num_programs(0)

    r_iota = jax.lax.broadcasted_iota(jnp.int32, (tws, nbatch), 0)
    c_iota = jax.lax.broadcasted_iota(jnp.int32, (tws, nbatch), 1)
    inv = c_iota * ncols + r_iota + jnp.int32(42)
    v0 = s * cbv

    rm = jnp.where(s == 0, jnp.full((tws, nbatch), -jnp.inf, jnp.float32),
                   best_val[...])
    ri = best_idx[...]

    for t in range(cbv // tws):
        x1i = inv + (v0 + t * tws)
        ng = _neg_gumbel_from_bits(_threefry_bits(x1i.astype(jnp.uint32)))
        vals = logits_ref[t * tws:(t + 1) * tws, :] - ng
        upd = vals > rm
        rm = jnp.where(upd, vals, rm)
        ri = jnp.where(upd, x1i, ri)

    best_val[...] = rm
    best_idx[...] = ri

    @pl.when(s == ns - 1)
    def _():
        m = jnp.max(rm, axis=0, keepdims=True)
        cand = jnp.where(rm == m, ri, jnp.int32(np.iinfo(np.int32).max))
        li = jnp.min(cand, axis=0, keepdims=True)
        b_iota = jax.lax.broadcasted_iota(jnp.int32, (1, nbatch), 1)
        out_ref[...] = li - b_iota * ncols - jnp.int32(42)


def kernel(logits):
    nrows, ncols = logits.shape
    lt = logits.T

    cbv = 4000 if ncols % 4000 == 0 else ncols
    tws = 40 if cbv % 40 == 0 else 8
    assert ncols % cbv == 0 and cbv % tws == 0

    out = pl.pallas_call(
        functools.partial(_sample_kernel, cbv=cbv, tws=tws, nbatch=nrows,
                          ncols=ncols),
        grid=(ncols // cbv,),
        in_specs=[pl.BlockSpec((cbv, nrows), lambda s: (s, 0))],
        out_specs=pl.BlockSpec((1, nrows), lambda s: (0, 0)),
        out_shape=jax.ShapeDtypeStruct((1, nrows), jnp.int32),
        scratch_shapes=[
            pltpu.VMEM((tws, nrows), jnp.float32),
            pltpu.VMEM((tws, nrows), jnp.int32),
        ],
    )(lt)
    return out.reshape(nrows).astype(jnp.int64)
